# baseline (speedup 1.0000x reference)
import jax
import jax.numpy as jnp
from jax import lax
from jax.experimental import pallas as pl
from jax.experimental.pallas import tpu as pltpu
from jax.experimental.pallas import tpu_sc as plsc

N = 32768
TARGET = int(0.99 * N)
W = N - TARGET + 1
CAP = 512
CAP_PAD = CAP + 16
HI_OFF = CAP - W
NROWS = 64
NC, NS = 2, 16
ROWS_PER_WORKER = NROWS // (NC * NS)
NV = N // 16
NB = 2048
NB1 = 256
NB2 = 8192

U = jnp.uint32


def _key_body(x_hbm, out_hbm, row_a, row_b, keys_v, h0, hl1, hh1, hl2, hh2,
              buf_lo, buf_hi, out_v, sem_a, sem_b):
    cid = lax.axis_index("c")
    sid = lax.axis_index("s")
    wid = sid * NC + cid
    ones = jnp.ones((16,), jnp.int32)
    zeros16 = jnp.zeros((16,), jnp.int32)
    iota = lax.iota(jnp.int32, 16)
    neg_inf = jnp.float32(-jnp.inf)
    pos_inf = jnp.float32(jnp.inf)

    def inv_to_float(k):
        top = k >> U(31)
        m2 = jnp.where(top == U(1), U(0x80000000), U(0xFFFFFFFF))
        return plsc.bitcast(k ^ m2, jnp.float32)

    def scan_hist(h, rank, nvec):
        def sbody(v, carry):
            found, base, tot = carry
            v = v.astype(jnp.int32)
            hv = h[pl.ds(v * 16, 16)]
            cin = plsc.cumsum(hv) + tot
            m = cin > rank
            has = jnp.max(m.astype(jnp.int32)) > 0
            lane = jnp.minimum(jnp.max(plsc.all_reduce_ffs(m)), 15)
            cumexcl = cin - hv
            baseval = jnp.max(jnp.where(iota == lane, cumexcl, jnp.int32(-(2**31))))
            upd = jnp.logical_and(found < 0, has)
            found = jnp.where(upd, v * 16 + lane, found)
            base = jnp.where(upd, baseval, base)
            return found, base, jnp.max(cin)

        f, b, _ = plsc.parallel_loop(
            0, nvec, unroll=4,
            carry=(jnp.int32(-1), jnp.int32(0), jnp.int32(0)))(sbody)
        return f, b

    def scan_hist2(h, rank_a, rank_b, nvec):
        def sbody(v, carry):
            fa, ba, fb, bb, tot = carry
            v = v.astype(jnp.int32)
            hv = h[pl.ds(v * 16, 16)]
            cin = plsc.cumsum(hv) + tot
            cumexcl = cin - hv

            def one(rank, found, base):
                m = cin > rank
                has = jnp.max(m.astype(jnp.int32)) > 0
                lane = jnp.minimum(jnp.max(plsc.all_reduce_ffs(m)), 15)
                bv = jnp.max(jnp.where(iota == lane, cumexcl,
                                       jnp.int32(-(2**31))))
                upd = jnp.logical_and(found < 0, has)
                return (jnp.where(upd, v * 16 + lane, found),
                        jnp.where(upd, bv, base))

            fa, ba = one(rank_a, fa, ba)
            fb, bb = one(rank_b, fb, bb)
            return fa, ba, fb, bb, jnp.max(cin)

        z = jnp.int32(0)
        fa, ba, fb, bb, _ = plsc.parallel_loop(
            0, nvec, unroll=4,
            carry=(jnp.int32(-1), z, jnp.int32(-1), z, z))(sbody)
        return fa, ba, fb, bb

    def bitonic_sort(buf):
        def ld(v):
            return buf[pl.ds(v * 16, 16)]

        def st(v, val):
            buf[pl.ds(v * 16, 16)] = val

        def vsort(v, descending):
            k = ld(v)
            ks, _ = plsc.sort_key_val(k, k, descending=descending)
            st(v, ks)

        for v in range(32):
            vsort(v, v % 2 == 1)
        for bk in (2, 4, 8, 16, 32):
            d = bk // 2
            while d >= 1:
                for base in range(0, 32, bk):
                    asc = (base // bk) % 2 == 0
                    for i0 in range(base, base + bk):
                        if (i0 - base) % (2 * d) < d:
                            va, vb = ld(i0), ld(i0 + d)
                            lo = jnp.minimum(va, vb)
                            hi = jnp.maximum(va, vb)
                            if asc:
                                st(i0, lo)
                                st(i0 + d, hi)
                            else:
                                st(i0, hi)
                                st(i0 + d, lo)
                d //= 2
            for v in range(32):
                vsort(v, descending=((v // bk) % 2 == 1) and bk < 32)

    def row_body(row, row_v):
        def clr(i):
            h0[pl.ds(i * 16, 16)] = zeros16

        plsc.parallel_loop(0, NB // 16, unroll=2)(clr)

        def clr1(i):
            hl1[pl.ds(i * 16, 16)] = zeros16
            hh1[pl.ds(i * 16, 16)] = zeros16

        plsc.parallel_loop(0, NB1 // 16, unroll=2)(clr1)

        def p1(i):
            f = row_v[pl.ds(i * 16, 16)]
            b = plsc.bitcast(f, U)
            negm = (b >> U(31)) * U(0xFFFFFFFF)
            key = b ^ (negm | U(0x80000000))
            keys_v[pl.ds(i * 16, 16)] = key
            bin0 = (key >> U(21)).astype(jnp.int32)
            cnts, lastm = plsc.scan_count(bin0)
            plsc.addupdate_scatter(h0, [bin0], cnts, mask=lastm)

        plsc.parallel_loop(0, NV, unroll=4)(p1)

        b_lo, base_lo, b_hi, base_hi = scan_hist2(
            h0, jnp.int32(W - 1), jnp.int32(TARGET - 1), NB // 16)
        pref_lo = b_lo.astype(U)
        pref_hi = b_hi.astype(U)
        r_lo = jnp.int32(W - 1) - base_lo
        r_hi = jnp.int32(TARGET - 1) - base_hi

        def ph(i):
            key = keys_v[pl.ds(i * 16, 16)]
            pk = key >> U(21)
            bin1 = ((key >> U(13)) & U(0xFF)).astype(jnp.int32)
            plsc.addupdate_scatter(hl1, [bin1], ones, mask=pk == pref_lo)
            plsc.addupdate_scatter(hh1, [bin1], ones, mask=pk == pref_hi)

        plsc.parallel_loop(0, NV, unroll=4)(ph)

        b_lo, base_lo = scan_hist(hl1, r_lo, NB1 // 16)
        b_hi, base_hi = scan_hist(hh1, r_hi, NB1 // 16)
        p19_lo = (pref_lo << U(8)) | b_lo.astype(U)
        p19_hi = (pref_hi << U(8)) | b_hi.astype(U)
        r_lo = r_lo - base_lo
        r_hi = r_hi - base_hi

        fill_lo = jnp.full((16,), U(0xFFFFFFFF))
        fill_hi = jnp.zeros((16,), U)

        def fill(i):
            buf_lo[pl.ds(i * 16, 16)] = fill_lo
            buf_hi[pl.ds(i * 16, 16)] = fill_hi

        plsc.parallel_loop(0, CAP_PAD // 16, unroll=3)(fill)

        def pg(i, carry):
            off_lo, off_hi = carry
            key = keys_v[pl.ds(i * 16, 16)]
            pk = key >> U(13)
            m_lo = pk <= p19_lo
            m_hi = pk >= p19_hi
            s_lo = jnp.minimum(off_lo, jnp.int32(CAP))
            s_hi = jnp.minimum(off_hi, jnp.int32(CAP))
            plsc.store_compressed(buf_lo.at[pl.ds(s_lo, 16)], key, mask=m_lo)
            plsc.store_compressed(buf_hi.at[pl.ds(s_hi, 16)], key, mask=m_hi)
            return (off_lo + jnp.sum(m_lo.astype(jnp.int32)),
                    off_hi + jnp.sum(m_hi.astype(jnp.int32)))

        cnt_lo, cnt_hi = plsc.parallel_loop(
            0, NV, unroll=4, carry=(jnp.int32(0), jnp.int32(0)))(pg)

        @pl.when(jnp.logical_or(cnt_lo > CAP, cnt_hi > CAP))
        def _slow():
            def clr2(i):
                hl2[pl.ds(i * 16, 16)] = zeros16
                hh2[pl.ds(i * 16, 16)] = zeros16

            plsc.parallel_loop(0, NB2 // 16, unroll=2)(clr2)

            def ph2(i):
                key = keys_v[pl.ds(i * 16, 16)]
                pk = key >> U(13)
                bin2 = (key & U(0x1FFF)).astype(jnp.int32)
                plsc.addupdate_scatter(hl2, [bin2], ones, mask=pk == p19_lo)
                plsc.addupdate_scatter(hh2, [bin2], ones, mask=pk == p19_hi)

            plsc.parallel_loop(0, NV, unroll=4)(ph2)

            b2_lo, _b = scan_hist(hl2, r_lo, NB2 // 16)
            b2_hi, _b2 = scan_hist(hh2, r_hi, NB2 // 16)
            k_lo = (p19_lo << U(13)) | b2_lo.astype(U)
            k_hi = (p19_hi << U(13)) | b2_hi.astype(U)
            k_lo_v = jnp.full((16,), k_lo, U)
            k_hi_v = jnp.full((16,), k_hi, U)

            def refill(i):
                buf_lo[pl.ds(i * 16, 16)] = k_lo_v
                buf_hi[pl.ds(i * 16, 16)] = k_hi_v

            plsc.parallel_loop(0, CAP_PAD // 16, unroll=3)(refill)

            def pg2(i, carry):
                off_lo, off_hi = carry
                key = keys_v[pl.ds(i * 16, 16)]
                m_lo = key < k_lo
                m_hi = key > k_hi
                plsc.store_compressed(
                    buf_lo.at[pl.ds(off_lo, 16)], key, mask=m_lo)
                plsc.store_compressed(
                    buf_hi.at[pl.ds(off_hi, 16)], key, mask=m_hi)
                return (off_lo + jnp.sum(m_lo.astype(jnp.int32)),
                        off_hi + jnp.sum(m_hi.astype(jnp.int32)))

            plsc.parallel_loop(
                0, NV, unroll=4, carry=(jnp.int32(0), jnp.int32(0)))(pg2)

        bitonic_sort(buf_lo)
        bitonic_sort(buf_hi)

        def am(i, carry):
            best, bl, br = carry
            i = i.astype(jnp.int32)
            lf = inv_to_float(buf_lo[pl.ds(i * 16, 16)])
            rf = inv_to_float(buf_hi[pl.ds(HI_OFF + i * 16, 16)])
            ln = rf - lf
            ln = jnp.where(i * 16 + iota < W, ln, pos_inf)
            vmin = jnp.min(ln)
            lane = jnp.minimum(jnp.max(plsc.all_reduce_ffs(ln == vmin)), 15)
            lval = jnp.max(jnp.where(iota == lane, lf, neg_inf))
            rval = jnp.max(jnp.where(iota == lane, rf, neg_inf))
            upd = vmin < best
            return (jnp.where(upd, vmin, best), jnp.where(upd, lval, bl),
                    jnp.where(upd, rval, br))

        _, best_l, best_r = plsc.parallel_loop(
            0, (W + 15) // 16, unroll=3,
            carry=(pos_inf, jnp.float32(0), jnp.float32(0)))(am)

        out_v[...] = jnp.where(iota == 0, best_l,
                               jnp.where(iota == 1, best_r, jnp.float32(0)))
        pltpu.sync_copy(out_v, out_hbm.at[row])

    row0 = wid * ROWS_PER_WORKER
    cp_a = pltpu.async_copy(x_hbm.at[row0], row_a, sem_a)
    cp_b = pltpu.async_copy(x_hbm.at[row0 + 1], row_b, sem_b)
    cp_a.wait()
    row_body(row0, row_a)
    cp_b.wait()
    row_body(row0 + 1, row_b)


@jax.jit
def kernel(x):
    mesh = plsc.VectorSubcoreMesh(core_axis_name="c", subcore_axis_name="s")
    run = pl.kernel(
        _key_body,
        out_type=jax.ShapeDtypeStruct((NROWS, 16), jnp.float32),
        mesh=mesh,
        compiler_params=pltpu.CompilerParams(needs_layout_passes=False),
        scratch_types=[
            pltpu.VMEM((N,), jnp.float32),
            pltpu.VMEM((N,), jnp.float32),
            pltpu.VMEM((N,), U),
            pltpu.VMEM((NB,), jnp.int32),
            pltpu.VMEM((NB1,), jnp.int32),
            pltpu.VMEM((NB1,), jnp.int32),
            pltpu.VMEM((NB2,), jnp.int32),
            pltpu.VMEM((NB2,), jnp.int32),
            pltpu.VMEM((CAP_PAD,), U),
            pltpu.VMEM((CAP_PAD,), U),
            pltpu.VMEM((16,), jnp.float32),
            pltpu.SemaphoreType.DMA,
            pltpu.SemaphoreType.DMA,
        ],
    )
    out = run(x)
    return (out[:, 0], out[:, 1])


# speedup vs baseline: 1.0206x; 1.0206x over previous
import jax
import jax.numpy as jnp
from jax import lax
from jax.experimental import pallas as pl
from jax.experimental.pallas import tpu as pltpu
from jax.experimental.pallas import tpu_sc as plsc

N = 32768
TARGET = int(0.99 * N)
W = N - TARGET + 1
CAP = 512
CAP_PAD = CAP + 16
HI_OFF = CAP - W
NROWS = 64
NC, NS = 2, 16
ROWS_PER_WORKER = NROWS // (NC * NS)
NV = N // 16
NB = 2048
NB1 = 256
NB2 = 8192

U = jnp.uint32


def _key_body(x_hbm, out_hbm, row_v, keys_v, h0, hl1, hh1, hl2, hh2,
              buf_lo, buf_hi, out_v):
    cid = lax.axis_index("c")
    sid = lax.axis_index("s")
    wid = sid * NC + cid
    ones = jnp.ones((16,), jnp.int32)
    zeros16 = jnp.zeros((16,), jnp.int32)
    iota = lax.iota(jnp.int32, 16)
    neg_inf = jnp.float32(-jnp.inf)
    pos_inf = jnp.float32(jnp.inf)

    def inv_to_float(k):
        top = k >> U(31)
        m2 = jnp.where(top == U(1), U(0x80000000), U(0xFFFFFFFF))
        return plsc.bitcast(k ^ m2, jnp.float32)

    def scan_hist(h, rank, nvec):
        def sbody(v, carry):
            found, base, tot = carry
            v = v.astype(jnp.int32)
            hv = h[pl.ds(v * 16, 16)]
            cin = plsc.cumsum(hv) + tot
            m = cin > rank
            has = jnp.max(m.astype(jnp.int32)) > 0
            lane = jnp.minimum(jnp.max(plsc.all_reduce_ffs(m)), 15)
            cumexcl = cin - hv
            baseval = jnp.max(jnp.where(iota == lane, cumexcl, jnp.int32(-(2**31))))
            upd = jnp.logical_and(found < 0, has)
            found = jnp.where(upd, v * 16 + lane, found)
            base = jnp.where(upd, baseval, base)
            return found, base, jnp.max(cin)

        f, b, _ = plsc.parallel_loop(
            0, nvec, unroll=4,
            carry=(jnp.int32(-1), jnp.int32(0), jnp.int32(0)))(sbody)
        return f, b

    def scan_hist2(h, rank_a, rank_b, nvec):
        def sbody(v, carry):
            fa, ba, fb, bb, tot = carry
            v = v.astype(jnp.int32)
            hv = h[pl.ds(v * 16, 16)]
            cin = plsc.cumsum(hv) + tot
            cumexcl = cin - hv

            def one(rank, found, base):
                m = cin > rank
                has = jnp.max(m.astype(jnp.int32)) > 0
                lane = jnp.minimum(jnp.max(plsc.all_reduce_ffs(m)), 15)
                bv = jnp.max(jnp.where(iota == lane, cumexcl,
                                       jnp.int32(-(2**31))))
                upd = jnp.logical_and(found < 0, has)
                return (jnp.where(upd, v * 16 + lane, found),
                        jnp.where(upd, bv, base))

            fa, ba = one(rank_a, fa, ba)
            fb, bb = one(rank_b, fb, bb)
            return fa, ba, fb, bb, jnp.max(cin)

        z = jnp.int32(0)
        fa, ba, fb, bb, _ = plsc.parallel_loop(
            0, nvec, unroll=4,
            carry=(jnp.int32(-1), z, jnp.int32(-1), z, z))(sbody)
        return fa, ba, fb, bb

    def bitonic_sort(buf):
        def ld(v):
            return buf[pl.ds(v * 16, 16)]

        def st(v, val):
            buf[pl.ds(v * 16, 16)] = val

        def vsort(v, descending):
            k = ld(v)
            ks, _ = plsc.sort_key_val(k, k, descending=descending)
            st(v, ks)

        for v in range(32):
            vsort(v, v % 2 == 1)
        for bk in (2, 4, 8, 16, 32):
            d = bk // 2
            while d >= 1:
                for base in range(0, 32, bk):
                    asc = (base // bk) % 2 == 0
                    for i0 in range(base, base + bk):
                        if (i0 - base) % (2 * d) < d:
                            va, vb = ld(i0), ld(i0 + d)
                            lo = jnp.minimum(va, vb)
                            hi = jnp.maximum(va, vb)
                            if asc:
                                st(i0, lo)
                                st(i0 + d, hi)
                            else:
                                st(i0, hi)
                                st(i0 + d, lo)
                d //= 2
            for v in range(32):
                vsort(v, descending=((v // bk) % 2 == 1) and bk < 32)

    def row_body(r, carry):
        row = wid * ROWS_PER_WORKER + r

        def clr(i):
            h0[pl.ds(i * 16, 16)] = zeros16

        plsc.parallel_loop(0, NB // 16, unroll=2)(clr)

        def clr1(i):
            hl1[pl.ds(i * 16, 16)] = zeros16
            hh1[pl.ds(i * 16, 16)] = zeros16

        plsc.parallel_loop(0, NB1 // 16, unroll=2)(clr1)
        pltpu.sync_copy(x_hbm.at[row], row_v)

        def p1(i):
            f = row_v[pl.ds(i * 16, 16)]
            b = plsc.bitcast(f, U)
            negm = (b >> U(31)) * U(0xFFFFFFFF)
            key = b ^ (negm | U(0x80000000))
            keys_v[pl.ds(i * 16, 16)] = key
            bin0 = (key >> U(21)).astype(jnp.int32)
            cnts, lastm = plsc.scan_count(bin0)
            plsc.addupdate_scatter(h0, [bin0], cnts, mask=lastm)

        plsc.parallel_loop(0, NV, unroll=4)(p1)

        b_lo, base_lo, b_hi, base_hi = scan_hist2(
            h0, jnp.int32(W - 1), jnp.int32(TARGET - 1), NB // 16)
        pref_lo = b_lo.astype(U)
        pref_hi = b_hi.astype(U)
        r_lo = jnp.int32(W - 1) - base_lo
        r_hi = jnp.int32(TARGET - 1) - base_hi

        def ph(i):
            key = keys_v[pl.ds(i * 16, 16)]
            pk = key >> U(21)
            bin1 = ((key >> U(13)) & U(0xFF)).astype(jnp.int32)
            plsc.addupdate_scatter(hl1, [bin1], ones, mask=pk == pref_lo)
            plsc.addupdate_scatter(hh1, [bin1], ones, mask=pk == pref_hi)

        plsc.parallel_loop(0, NV, unroll=4)(ph)

        b_lo, base_lo = scan_hist(hl1, r_lo, NB1 // 16)
        b_hi, base_hi = scan_hist(hh1, r_hi, NB1 // 16)
        p19_lo = (pref_lo << U(8)) | b_lo.astype(U)
        p19_hi = (pref_hi << U(8)) | b_hi.astype(U)
        r_lo = r_lo - base_lo
        r_hi = r_hi - base_hi

        fill_lo = jnp.full((16,), U(0xFFFFFFFF))
        fill_hi = jnp.zeros((16,), U)

        def fill(i):
            buf_lo[pl.ds(i * 16, 16)] = fill_lo
            buf_hi[pl.ds(i * 16, 16)] = fill_hi

        plsc.parallel_loop(0, CAP_PAD // 16, unroll=3)(fill)

        def pg(i, carry):
            off_lo, off_hi = carry
            key = keys_v[pl.ds(i * 16, 16)]
            pk = key >> U(13)
            m_lo = pk <= p19_lo
            m_hi = pk >= p19_hi
            s_lo = jnp.minimum(off_lo, jnp.int32(CAP))
            s_hi = jnp.minimum(off_hi, jnp.int32(CAP))
            plsc.store_compressed(buf_lo.at[pl.ds(s_lo, 16)], key, mask=m_lo)
            plsc.store_compressed(buf_hi.at[pl.ds(s_hi, 16)], key, mask=m_hi)
            return (off_lo + jnp.sum(m_lo.astype(jnp.int32)),
                    off_hi + jnp.sum(m_hi.astype(jnp.int32)))

        cnt_lo, cnt_hi = plsc.parallel_loop(
            0, NV, unroll=4, carry=(jnp.int32(0), jnp.int32(0)))(pg)

        @pl.when(jnp.logical_or(cnt_lo > CAP, cnt_hi > CAP))
        def _slow():
            def clr2(i):
                hl2[pl.ds(i * 16, 16)] = zeros16
                hh2[pl.ds(i * 16, 16)] = zeros16

            plsc.parallel_loop(0, NB2 // 16, unroll=2)(clr2)

            def ph2(i):
                key = keys_v[pl.ds(i * 16, 16)]
                pk = key >> U(13)
                bin2 = (key & U(0x1FFF)).astype(jnp.int32)
                plsc.addupdate_scatter(hl2, [bin2], ones, mask=pk == p19_lo)
                plsc.addupdate_scatter(hh2, [bin2], ones, mask=pk == p19_hi)

            plsc.parallel_loop(0, NV, unroll=4)(ph2)

            b2_lo, _b = scan_hist(hl2, r_lo, NB2 // 16)
            b2_hi, _b2 = scan_hist(hh2, r_hi, NB2 // 16)
            k_lo = (p19_lo << U(13)) | b2_lo.astype(U)
            k_hi = (p19_hi << U(13)) | b2_hi.astype(U)
            k_lo_v = jnp.full((16,), k_lo, U)
            k_hi_v = jnp.full((16,), k_hi, U)

            def refill(i):
                buf_lo[pl.ds(i * 16, 16)] = k_lo_v
                buf_hi[pl.ds(i * 16, 16)] = k_hi_v

            plsc.parallel_loop(0, CAP_PAD // 16, unroll=3)(refill)

            def pg2(i, carry):
                off_lo, off_hi = carry
                key = keys_v[pl.ds(i * 16, 16)]
                m_lo = key < k_lo
                m_hi = key > k_hi
                plsc.store_compressed(
                    buf_lo.at[pl.ds(off_lo, 16)], key, mask=m_lo)
                plsc.store_compressed(
                    buf_hi.at[pl.ds(off_hi, 16)], key, mask=m_hi)
                return (off_lo + jnp.sum(m_lo.astype(jnp.int32)),
                        off_hi + jnp.sum(m_hi.astype(jnp.int32)))

            plsc.parallel_loop(
                0, NV, unroll=4, carry=(jnp.int32(0), jnp.int32(0)))(pg2)

        bitonic_sort(buf_lo)
        bitonic_sort(buf_hi)

        def am(i, carry):
            best, bl, br = carry
            i = i.astype(jnp.int32)
            lf = inv_to_float(buf_lo[pl.ds(i * 16, 16)])
            rf = inv_to_float(buf_hi[pl.ds(HI_OFF + i * 16, 16)])
            ln = rf - lf
            ln = jnp.where(i * 16 + iota < W, ln, pos_inf)
            vmin = jnp.min(ln)
            lane = jnp.minimum(jnp.max(plsc.all_reduce_ffs(ln == vmin)), 15)
            lval = jnp.max(jnp.where(iota == lane, lf, neg_inf))
            rval = jnp.max(jnp.where(iota == lane, rf, neg_inf))
            upd = vmin < best
            return (jnp.where(upd, vmin, best), jnp.where(upd, lval, bl),
                    jnp.where(upd, rval, br))

        _, best_l, best_r = plsc.parallel_loop(
            0, (W + 15) // 16, unroll=3,
            carry=(pos_inf, jnp.float32(0), jnp.float32(0)))(am)

        out_v[...] = jnp.where(iota == 0, best_l,
                               jnp.where(iota == 1, best_r, jnp.float32(0)))
        pltpu.sync_copy(out_v, out_hbm.at[row])
        return carry

    lax.fori_loop(0, ROWS_PER_WORKER, row_body, 0)


@jax.jit
def kernel(x):
    mesh = plsc.VectorSubcoreMesh(core_axis_name="c", subcore_axis_name="s")
    run = pl.kernel(
        _key_body,
        out_type=jax.ShapeDtypeStruct((NROWS, 16), jnp.float32),
        mesh=mesh,
        compiler_params=pltpu.CompilerParams(needs_layout_passes=False),
        scratch_types=[
            pltpu.VMEM((N,), jnp.float32),
            pltpu.VMEM((N,), U),
            pltpu.VMEM((NB,), jnp.int32),
            pltpu.VMEM((NB1,), jnp.int32),
            pltpu.VMEM((NB1,), jnp.int32),
            pltpu.VMEM((NB2,), jnp.int32),
            pltpu.VMEM((NB2,), jnp.int32),
            pltpu.VMEM((CAP_PAD,), U),
            pltpu.VMEM((CAP_PAD,), U),
            pltpu.VMEM((16,), jnp.float32),
        ],
    )
    out = run(x)
    return (out[:, 0], out[:, 1])
